# baseline (speedup 1.0000x reference)
import functools

import jax
import jax.numpy as jnp
from jax import lax
from jax.experimental import pallas as pl
from jax.experimental.pallas import tpu as pltpu
from jax.experimental.pallas import tpu_sc as plsc

VOCAB = 1000000
EMBED_DIM = 64
BATCH = 4096
SEQ = 200

_NC = 2
_NS = 16
_NW = _NC * _NS

_LANES = 128
_PAIRS = VOCAB // 2
_NTILE_A = 7808
_SWW = 256
_SPW = _NTILE_A * 128 // _SWW // _NW
_TAIL_ROWS = VOCAB - _NTILE_A * 128
_TAIL_PAIRS = _TAIL_ROWS // 2

_NBUF = 4
_NSEQ = SEQ


def _relayout_kernel(table_t_hbm, tail_hbm, scratch_hbm,
                     src0, src1, dst0, dst1, tailv,
                     rsem0, rsem1, wsem0, wsem1):
    srcs = (src0, src1)
    dsts = (dst0, dst1)
    rsems = (rsem0, rsem1)
    wsems = (wsem0, wsem1)
    wid = lax.axis_index("s") * _NC + lax.axis_index("c")
    start = wid * _SPW
    iota = lax.iota(jnp.int32, 16)

    cA = tuple((16 * g + iota) >> 1 for g in range(_SWW // 16))
    cB = tuple(((16 * g + iota) & 1) * 64 for g in range(_SWW // 16))

    def rd(sw, p, k):
        return pltpu.make_async_copy(
            table_t_hbm.at[pl.ds(8 * p, 8), pl.ds((start + sw) * _SWW, _SWW)],
            srcs[k], rsems[k])

    def wr(sw, z):
        return pltpu.make_async_copy(
            dsts[z], scratch_hbm.at[pl.ds((start + sw) * (_SWW // 2), _SWW // 2)],
            wsems[z])

    def do_plane(sw, p, z):
        k = p % 2
        rd(sw, p, k).wait()

        @plsc.parallel_loop(0, 8, 1, unroll=2)
        def _(q):
            for g in range(_SWW // 16):
                val = srcs[k][q, pl.ds(16 * g, 16)]
                plsc.store_scatter(dsts[z], [cA[g], cB[g] + (8 * p + q)], val)

        if p < 6:
            rd(sw, p + 2, k).start()
        else:
            @pl.when(sw < _SPW - 1)
            def _():
                rd(sw + 1, p - 6, k).start()

    rd(0, 0, 0).start()
    rd(0, 1, 1).start()

    def body(u, carry):
        for z in range(2):
            sw = 2 * u + z

            @pl.when(u > 0)
            def _():
                wr(sw - 2, z).wait()

            for p in range(8):
                do_plane(sw, p, z)
            wr(sw, z).start()
        return carry

    lax.fori_loop(0, _SPW // 2, body, 0)
    wr(_SPW - 2, 0).wait()
    wr(_SPW - 1, 1).wait()

    @pl.when(wid == _NW - 1)
    def _():
        pltpu.sync_copy(tail_hbm, tailv)
        pltpu.sync_copy(tailv, scratch_hbm.at[pl.ds(_NTILE_A * 64, _TAIL_PAIRS)])


def _gather_kernel(scratch_hbm, idxt_hbm, out_hbm,
                   idx_v, i0, i1, i2, i3, g0, g1, g2, g3, t0, t1,
                   gs0, gs1, gs2, gs3, ts0, ts1):
    idx2 = (i0, i1, i2, i3)
    gbuf = (g0, g1, g2, g3)
    tbuf = (t0, t1)
    gsems = (gs0, gs1, gs2, gs3)
    tsems = (ts0, ts1)
    wid = lax.axis_index("s") * _NC + lax.axis_index("c")
    iota = lax.iota(jnp.int32, 16)
    idxL = tuple(16 * g + iota for g in range(8))

    pltpu.sync_copy(idxt_hbm.at[:, pl.ds(wid * 128, 128)], idx_v)

    def prep_and_fire(i, b):
        for c in range(8):
            idx2[b][pl.ds(16 * c, 16)] = lax.shift_right_logical(
                idx_v[i, pl.ds(16 * c, 16)], 1)
        pltpu.make_async_copy(scratch_hbm.at[idx2[b]], gbuf[b], gsems[b]).start()

    def out_tile(i, tb):
        for p in range(8):
            pltpu.make_async_copy(tbuf[tb].at[pl.ds(8 * p, 8), :],
                                  out_hbm.at[i, p, wid], tsems[tb]).start()

    def drain_out(i, tb):
        for p in range(8):
            pltpu.make_async_copy(tbuf[tb].at[pl.ds(8 * p, 8), :],
                                  out_hbm.at[i, p, wid], tsems[tb]).wait()

    for b in range(_NBUF):
        prep_and_fire(b, b)

    def body(t, carry):
        for b in range(_NBUF):
            i = _NBUF * t + b
            tb = b % 2
            pltpu.make_async_copy(scratch_hbm.at[idx2[b]], gbuf[b], gsems[b]).wait()

            @pl.when(t < (_NSEQ // _NBUF) - 1)
            def _():
                for c in range(8):
                    idx2[b][pl.ds(16 * c, 16)] = lax.shift_right_logical(
                        idx_v[i + _NBUF, pl.ds(16 * c, 16)], 1)

            if b >= 2:
                drain_out(i - 2, tb)
            else:
                @pl.when(t > 0)
                def _():
                    drain_out(i - 2, tb)

            halves = []
            for g in range(8):
                halves.append(
                    lax.shift_left(jnp.bitwise_and(idx_v[i, pl.ds(16 * g, 16)], 1), 6))

            @plsc.parallel_loop(0, EMBED_DIM, 1, unroll=4)
            def _(d):
                for g in range(8):
                    val = plsc.load_gather(gbuf[b], [idxL[g], halves[g] + d])
                    tbuf[tb][d, pl.ds(16 * g, 16)] = val

            @pl.when(t < (_NSEQ // _NBUF) - 1)
            def _():
                pltpu.make_async_copy(scratch_hbm.at[idx2[b]], gbuf[b],
                                      gsems[b]).start()
            out_tile(i, tb)
        return carry

    lax.fori_loop(0, _NSEQ // _NBUF, body, 0)
    drain_out(_NSEQ - 2, 0)
    drain_out(_NSEQ - 1, 1)


@jax.jit
def kernel(input_sentence, table):
    mesh = plsc.VectorSubcoreMesh(core_axis_name="c", subcore_axis_name="s")
    cparams = pltpu.CompilerParams(use_tc_tiling_on_sc=True, needs_layout_passes=False)

    table_t = table.T
    tail = table[_NTILE_A * 128:].reshape(_TAIL_PAIRS, _LANES)
    idxt = input_sentence.T.astype(jnp.int32)

    scratch = pl.kernel(
        _relayout_kernel,
        out_type=jax.ShapeDtypeStruct((_PAIRS, _LANES), jnp.float32),
        mesh=mesh,
        scratch_types=(
            [pltpu.VMEM((8, _SWW), jnp.float32) for _ in range(2)]
            + [pltpu.VMEM((_SWW // 2, _LANES), jnp.float32) for _ in range(2)]
            + [pltpu.VMEM((_TAIL_PAIRS, _LANES), jnp.float32)]
            + [pltpu.SemaphoreType.DMA for _ in range(4)]
        ),
        compiler_params=cparams,
    )(table_t, tail)

    out5d = pl.kernel(
        _gather_kernel,
        out_type=jax.ShapeDtypeStruct((SEQ, 8, 32, 8, _LANES), jnp.float32),
        mesh=mesh,
        scratch_types=(
            [pltpu.VMEM((SEQ, _LANES), jnp.int32)]
            + [pltpu.VMEM((_LANES,), jnp.int32) for _ in range(_NBUF)]
            + [pltpu.VMEM((_LANES, _LANES), jnp.float32) for _ in range(_NBUF)]
            + [pltpu.VMEM((EMBED_DIM, _LANES), jnp.float32) for _ in range(2)]
            + [pltpu.SemaphoreType.DMA for _ in range(_NBUF + 2)]
        ),
        compiler_params=cparams,
    )(scratch, idxt)

    return out5d.transpose(2, 4, 0, 1, 3).reshape(BATCH, SEQ, EMBED_DIM)


# speedup vs baseline: 1.9969x; 1.9969x over previous
import functools

import jax
import jax.numpy as jnp
from jax import lax
from jax.experimental import pallas as pl
from jax.experimental.pallas import tpu as pltpu
from jax.experimental.pallas import tpu_sc as plsc

VOCAB = 1000000
EMBED_DIM = 64
BATCH = 4096
SEQ = 200

_NC = 2
_NS = 16
_NW = _NC * _NS

_LANES = 128
_PAIRS = VOCAB // 2
_NTILE_A = 7808
_SWW = 256
_SPW = _NTILE_A * 128 // _SWW // _NW
_TAIL_ROWS = VOCAB - _NTILE_A * 128
_TAIL_PAIRS = _TAIL_ROWS // 2

_NBUF = 4
_NSEQ = SEQ


def _relayout_kernel(table_t_hbm, tail_hbm, scratch_hbm,
                     src0, src1, dst0, dst1, tailv,
                     rsem0, rsem1, wsem0, wsem1):
    srcs = (src0, src1)
    dsts = (dst0, dst1)
    rsems = (rsem0, rsem1)
    wsems = (wsem0, wsem1)
    wid = lax.axis_index("s") * _NC + lax.axis_index("c")
    start = wid * _SPW
    iota = lax.iota(jnp.int32, 16)

    cA = tuple((16 * g + iota) >> 1 for g in range(_SWW // 16))
    cRot = tuple((((16 * g + iota) >> 1) & 7) + (((16 * g + iota) & 1) << 3)
                 for g in range(_SWW // 16))
    cBase = tuple(((16 * g + iota) & 1) * 64 for g in range(_SWW // 16))

    def rd(sw, p, k):
        return pltpu.make_async_copy(
            table_t_hbm.at[pl.ds(8 * p, 8), pl.ds((start + sw) * _SWW, _SWW)],
            srcs[k], rsems[k])

    def wr(sw, z):
        return pltpu.make_async_copy(
            dsts[z], scratch_hbm.at[pl.ds((start + sw) * (_SWW // 2), _SWW // 2)],
            wsems[z])

    def do_plane(sw, p, z):
        k = p % 2
        rd(sw, p, k).wait()

        @plsc.parallel_loop(0, 8, 1, unroll=2)
        def _(q):
            for g in range(_SWW // 16):
                val = srcs[k][q, pl.ds(16 * g, 16)]
                plsc.store_scatter(
                    dsts[z],
                    [cA[g], cBase[g] + ((cRot[g] + (8 * p + q)) & 63)], val)

        if p < 6:
            rd(sw, p + 2, k).start()
        else:
            @pl.when(sw < _SPW - 1)
            def _():
                rd(sw + 1, p - 6, k).start()

    rd(0, 0, 0).start()
    rd(0, 1, 1).start()

    def body(u, carry):
        for z in range(2):
            sw = 2 * u + z

            @pl.when(u > 0)
            def _():
                wr(sw - 2, z).wait()

            for p in range(8):
                do_plane(sw, p, z)
            wr(sw, z).start()
        return carry

    lax.fori_loop(0, _SPW // 2, body, 0)
    wr(_SPW - 2, 0).wait()
    wr(_SPW - 1, 1).wait()

    @pl.when(wid == _NW - 1)
    def _():
        pltpu.sync_copy(tail_hbm, tailv)
        pltpu.sync_copy(tailv, scratch_hbm.at[pl.ds(_NTILE_A * 64, _TAIL_PAIRS)])


def _gather_kernel(scratch_hbm, idxt_hbm, out_hbm,
                   idx_v, i0, i1, i2, i3, g0, g1, g2, g3, t0, t1,
                   gs0, gs1, gs2, gs3, ts0, ts1):
    idx2 = (i0, i1, i2, i3)
    gbuf = (g0, g1, g2, g3)
    tbuf = (t0, t1)
    gsems = (gs0, gs1, gs2, gs3)
    tsems = (ts0, ts1)
    wid = lax.axis_index("s") * _NC + lax.axis_index("c")
    iota = lax.iota(jnp.int32, 16)
    idxL = tuple(16 * g + iota for g in range(8))

    pltpu.sync_copy(idxt_hbm.at[:, pl.ds(wid * 128, 128)], idx_v)

    def prep_and_fire(i, b):
        for c in range(8):
            idx2[b][pl.ds(16 * c, 16)] = lax.shift_right_logical(
                idx_v[i, pl.ds(16 * c, 16)], 1)
        pltpu.make_async_copy(scratch_hbm.at[idx2[b]], gbuf[b], gsems[b]).start()

    def out_tile(i, tb):
        for p in range(8):
            pltpu.make_async_copy(tbuf[tb].at[pl.ds(8 * p, 8), :],
                                  out_hbm.at[i, p, wid], tsems[tb]).start()

    def drain_out(i, tb):
        for p in range(8):
            pltpu.make_async_copy(tbuf[tb].at[pl.ds(8 * p, 8), :],
                                  out_hbm.at[i, p, wid], tsems[tb]).wait()

    for b in range(_NBUF):
        prep_and_fire(b, b)

    def body(t, carry):
        for b in range(_NBUF):
            i = _NBUF * t + b
            tb = b % 2
            pltpu.make_async_copy(scratch_hbm.at[idx2[b]], gbuf[b], gsems[b]).wait()

            @pl.when(t < (_NSEQ // _NBUF) - 1)
            def _():
                for c in range(8):
                    idx2[b][pl.ds(16 * c, 16)] = lax.shift_right_logical(
                        idx_v[i + _NBUF, pl.ds(16 * c, 16)], 1)

            if b >= 2:
                drain_out(i - 2, tb)
            else:
                @pl.when(t > 0)
                def _():
                    drain_out(i - 2, tb)

            rots = []
            bases = []
            for g in range(8):
                rv = idx_v[i, pl.ds(16 * g, 16)]
                h = jnp.bitwise_and(rv, 1)
                rots.append(jnp.bitwise_and(lax.shift_right_logical(rv, 1), 7)
                            + lax.shift_left(h, 3))
                bases.append(lax.shift_left(h, 6))

            @plsc.parallel_loop(0, EMBED_DIM, 1, unroll=4)
            def _(d):
                for g in range(8):
                    val = plsc.load_gather(
                        gbuf[b], [idxL[g], bases[g] + ((rots[g] + d) & 63)])
                    tbuf[tb][d, pl.ds(16 * g, 16)] = val

            @pl.when(t < (_NSEQ // _NBUF) - 1)
            def _():
                pltpu.make_async_copy(scratch_hbm.at[idx2[b]], gbuf[b],
                                      gsems[b]).start()
            out_tile(i, tb)
        return carry

    lax.fori_loop(0, _NSEQ // _NBUF, body, 0)
    drain_out(_NSEQ - 2, 0)
    drain_out(_NSEQ - 1, 1)


@jax.jit
def kernel(input_sentence, table):
    mesh = plsc.VectorSubcoreMesh(core_axis_name="c", subcore_axis_name="s")
    cparams = pltpu.CompilerParams(use_tc_tiling_on_sc=True, needs_layout_passes=False)

    table_t = table.T
    pp = jnp.arange(_TAIL_PAIRS)[:, None]
    j = jnp.arange(_LANES)[None, :]
    h = j >> 6
    d = (j - 64 * h - ((pp & 7) + 8 * h)) % 64
    tail = table[_NTILE_A * 128 + 2 * pp + h, d]
    idxt = input_sentence.T.astype(jnp.int32)

    scratch = pl.kernel(
        _relayout_kernel,
        out_type=jax.ShapeDtypeStruct((_PAIRS, _LANES), jnp.float32),
        mesh=mesh,
        scratch_types=(
            [pltpu.VMEM((8, _SWW), jnp.float32) for _ in range(2)]
            + [pltpu.VMEM((_SWW // 2, _LANES), jnp.float32) for _ in range(2)]
            + [pltpu.VMEM((_TAIL_PAIRS, _LANES), jnp.float32)]
            + [pltpu.SemaphoreType.DMA for _ in range(4)]
        ),
        compiler_params=cparams,
    )(table_t, tail)

    out5d = pl.kernel(
        _gather_kernel,
        out_type=jax.ShapeDtypeStruct((SEQ, 8, 32, 8, _LANES), jnp.float32),
        mesh=mesh,
        scratch_types=(
            [pltpu.VMEM((SEQ, _LANES), jnp.int32)]
            + [pltpu.VMEM((_LANES,), jnp.int32) for _ in range(_NBUF)]
            + [pltpu.VMEM((_LANES, _LANES), jnp.float32) for _ in range(_NBUF)]
            + [pltpu.VMEM((EMBED_DIM, _LANES), jnp.float32) for _ in range(2)]
            + [pltpu.SemaphoreType.DMA for _ in range(_NBUF + 2)]
        ),
        compiler_params=cparams,
    )(scratch, idxt)

    return out5d.transpose(2, 4, 0, 1, 3).reshape(BATCH, SEQ, EMBED_DIM)


# speedup vs baseline: 2.0036x; 1.0033x over previous
import functools

import jax
import jax.numpy as jnp
from jax import lax
from jax.experimental import pallas as pl
from jax.experimental.pallas import tpu as pltpu
from jax.experimental.pallas import tpu_sc as plsc

VOCAB = 1000000
EMBED_DIM = 64
BATCH = 4096
SEQ = 200

_NC = 2
_NS = 16
_NW = _NC * _NS

_LANES = 128
_PAIRS = VOCAB // 2
_NTILE_A = 7808
_SWW = 256
_SPW = _NTILE_A * 128 // _SWW // _NW
_TAIL_ROWS = VOCAB - _NTILE_A * 128
_TAIL_PAIRS = _TAIL_ROWS // 2

_NBUF = 4
_NSEQ = SEQ


def _relayout_kernel(table_t_hbm, tail_hbm, scratch_hbm,
                     src0, src1, dst0, dst1, tailv,
                     rsem0, rsem1, wsem0, wsem1):
    srcs = (src0, src1)
    dsts = (dst0, dst1)
    rsems = (rsem0, rsem1)
    wsems = (wsem0, wsem1)
    wid = lax.axis_index("s") * _NC + lax.axis_index("c")
    start = wid * _SPW
    iota = lax.iota(jnp.int32, 16)

    cA = tuple((16 * g + iota) >> 1 for g in range(_SWW // 16))
    cRot = tuple(((((16 * g + iota) >> 1) & 7) << 3) + (((16 * g + iota) & 1) << 2)
                 for g in range(_SWW // 16))
    cBase = tuple(((16 * g + iota) & 1) * 64 for g in range(_SWW // 16))

    def rd(sw, p, k):
        return pltpu.make_async_copy(
            table_t_hbm.at[pl.ds(8 * p, 8), pl.ds((start + sw) * _SWW, _SWW)],
            srcs[k], rsems[k])

    def wr(sw, z):
        return pltpu.make_async_copy(
            dsts[z], scratch_hbm.at[pl.ds((start + sw) * (_SWW // 2), _SWW // 2)],
            wsems[z])

    def do_plane(sw, p, z):
        k = p % 2
        rd(sw, p, k).wait()

        @plsc.parallel_loop(0, 8, 1, unroll=2)
        def _(q):
            for g in range(_SWW // 16):
                val = srcs[k][q, pl.ds(16 * g, 16)]
                plsc.store_scatter(
                    dsts[z],
                    [cA[g], cBase[g] + ((cRot[g] + (8 * p + q)) & 63)], val)

        if p < 6:
            rd(sw, p + 2, k).start()
        else:
            @pl.when(sw < _SPW - 1)
            def _():
                rd(sw + 1, p - 6, k).start()

    rd(0, 0, 0).start()
    rd(0, 1, 1).start()

    def body(u, carry):
        for z in range(2):
            sw = 2 * u + z

            @pl.when(u > 0)
            def _():
                wr(sw - 2, z).wait()

            for p in range(8):
                do_plane(sw, p, z)
            wr(sw, z).start()
        return carry

    lax.fori_loop(0, _SPW // 2, body, 0)
    wr(_SPW - 2, 0).wait()
    wr(_SPW - 1, 1).wait()

    @pl.when(wid == _NW - 1)
    def _():
        pltpu.sync_copy(tail_hbm, tailv)
        pltpu.sync_copy(tailv, scratch_hbm.at[pl.ds(_NTILE_A * 64, _TAIL_PAIRS)])


def _gather_kernel(scratch_hbm, idxt_hbm, out_hbm,
                   idx_v, i0, i1, i2, i3, g0, g1, g2, g3, t0, t1,
                   gs0, gs1, gs2, gs3, ts0, ts1):
    idx2 = (i0, i1, i2, i3)
    gbuf = (g0, g1, g2, g3)
    tbuf = (t0, t1)
    gsems = (gs0, gs1, gs2, gs3)
    tsems = (ts0, ts1)
    wid = lax.axis_index("s") * _NC + lax.axis_index("c")
    iota = lax.iota(jnp.int32, 16)
    idxL = tuple(16 * g + iota for g in range(8))

    pltpu.sync_copy(idxt_hbm.at[:, pl.ds(wid * 128, 128)], idx_v)

    def prep_and_fire(i, b):
        for c in range(8):
            idx2[b][pl.ds(16 * c, 16)] = lax.shift_right_logical(
                idx_v[i, pl.ds(16 * c, 16)], 1)
        pltpu.make_async_copy(scratch_hbm.at[idx2[b]], gbuf[b], gsems[b]).start()

    def out_tile(i, tb):
        for p in range(8):
            pltpu.make_async_copy(tbuf[tb].at[pl.ds(8 * p, 8), :],
                                  out_hbm.at[i, p, wid], tsems[tb]).start()

    def drain_out(i, tb):
        for p in range(8):
            pltpu.make_async_copy(tbuf[tb].at[pl.ds(8 * p, 8), :],
                                  out_hbm.at[i, p, wid], tsems[tb]).wait()

    for b in range(_NBUF):
        prep_and_fire(b, b)

    def body(t, carry):
        for b in range(_NBUF):
            i = _NBUF * t + b
            tb = b % 2
            pltpu.make_async_copy(scratch_hbm.at[idx2[b]], gbuf[b], gsems[b]).wait()

            @pl.when(t < (_NSEQ // _NBUF) - 1)
            def _():
                for c in range(8):
                    idx2[b][pl.ds(16 * c, 16)] = lax.shift_right_logical(
                        idx_v[i + _NBUF, pl.ds(16 * c, 16)], 1)

            if b >= 2:
                drain_out(i - 2, tb)
            else:
                @pl.when(t > 0)
                def _():
                    drain_out(i - 2, tb)

            rots = []
            bases = []
            for g in range(8):
                rv = idx_v[i, pl.ds(16 * g, 16)]
                h = jnp.bitwise_and(rv, 1)
                rots.append(
                    lax.shift_left(jnp.bitwise_and(lax.shift_right_logical(rv, 1), 7), 3)
                    + lax.shift_left(h, 2))
                bases.append(lax.shift_left(h, 6))

            @plsc.parallel_loop(0, EMBED_DIM, 1, unroll=4)
            def _(d):
                for g in range(8):
                    val = plsc.load_gather(
                        gbuf[b], [idxL[g], bases[g] + ((rots[g] + d) & 63)])
                    tbuf[tb][d, pl.ds(16 * g, 16)] = val

            @pl.when(t < (_NSEQ // _NBUF) - 1)
            def _():
                pltpu.make_async_copy(scratch_hbm.at[idx2[b]], gbuf[b],
                                      gsems[b]).start()
            out_tile(i, tb)
        return carry

    lax.fori_loop(0, _NSEQ // _NBUF, body, 0)
    drain_out(_NSEQ - 2, 0)
    drain_out(_NSEQ - 1, 1)


@jax.jit
def kernel(input_sentence, table):
    mesh = plsc.VectorSubcoreMesh(core_axis_name="c", subcore_axis_name="s")
    cparams = pltpu.CompilerParams(use_tc_tiling_on_sc=True, needs_layout_passes=False)

    table_t = table.T
    pp = jnp.arange(_TAIL_PAIRS)[:, None]
    j = jnp.arange(_LANES)[None, :]
    h = j >> 6
    d = (j - 64 * h - (8 * (pp & 7) + 4 * h)) % 64
    tail = table[_NTILE_A * 128 + 2 * pp + h, d]
    idxt = input_sentence.T.astype(jnp.int32)

    scratch = pl.kernel(
        _relayout_kernel,
        out_type=jax.ShapeDtypeStruct((_PAIRS, _LANES), jnp.float32),
        mesh=mesh,
        scratch_types=(
            [pltpu.VMEM((8, _SWW), jnp.float32) for _ in range(2)]
            + [pltpu.VMEM((_SWW // 2, _LANES), jnp.float32) for _ in range(2)]
            + [pltpu.VMEM((_TAIL_PAIRS, _LANES), jnp.float32)]
            + [pltpu.SemaphoreType.DMA for _ in range(4)]
        ),
        compiler_params=cparams,
    )(table_t, tail)

    out5d = pl.kernel(
        _gather_kernel,
        out_type=jax.ShapeDtypeStruct((SEQ, 8, 32, 8, _LANES), jnp.float32),
        mesh=mesh,
        scratch_types=(
            [pltpu.VMEM((SEQ, _LANES), jnp.int32)]
            + [pltpu.VMEM((_LANES,), jnp.int32) for _ in range(_NBUF)]
            + [pltpu.VMEM((_LANES, _LANES), jnp.float32) for _ in range(_NBUF)]
            + [pltpu.VMEM((EMBED_DIM, _LANES), jnp.float32) for _ in range(2)]
            + [pltpu.SemaphoreType.DMA for _ in range(_NBUF + 2)]
        ),
        compiler_params=cparams,
    )(scratch, idxt)

    return out5d.transpose(2, 4, 0, 1, 3).reshape(BATCH, SEQ, EMBED_DIM)
